# baseline (speedup 1.0000x reference)
import math

import jax
import jax.numpy as jnp
from jax.experimental import pallas as pl
from jax.experimental.pallas import tpu as pltpu

_HIDDEN_DIM = 1024
_WORKSPACE_DIM = 256
_MEMORY_TOKEN_DIM = 1024
_TEMPERATURE = 0.25
_RUNGS = [
    (8, 96, 192, 0.55, 0.6, 0.5),
    (6, 128, 256, 0.55, 0.6, 0.5),
    (4, 160, 320, 0.55, 0.6, 0.5),
]
_CAD_PRIOR = 1.0 / (1.0 + math.exp(1.0))

_CHUNK = 512
_NSTEP = 4096 // _CHUNK
_GATE_HID = 384


def _pack_order():
    items = [("ev_b1", 1024)]
    for r in range(len(_RUNGS)):
        for n in ("sp_b", "sp_g", "sp_bb", "st_b", "st_g", "st_bb", "ro_b2"):
            items.append((f"r{r}_{n}", 1024))
    items += [("lv_b1", 512), ("lw", 512), ("lc", 512)]
    for r in range(len(_RUNGS)):
        for n in ("k_b1", "v_b1", "ro_b1"):
            items.append((f"r{r}_{n}", 512))
    for r in range(len(_RUNGS)):
        for g in range(3):
            items += [(f"r{r}g{g}_b1", _GATE_HID), (f"r{r}g{g}_w2", _GATE_HID)]
    items += [("ev_b2", 256), ("lv_b2", 256), ("lw_b", 1), ("lc_b", 1)]
    for r in range(len(_RUNGS)):
        for g in range(3):
            items.append((f"r{r}g{g}_b2", 1))
    for r, (_ns, kd, vd, *_t) in enumerate(_RUNGS):
        items += [(f"r{r}_k_b2", kd), (f"r{r}_v_b2", vd)]
    return items


_OFFS = {}
_o = 0
for _n, _w in _pack_order():
    _OFFS[_n] = (_o, _w)
    _o += _w
_PACK_LEN = _o


def _big_plan():
    plan = [(1, _PACK_LEN),
            (2 * _HIDDEN_DIM, _HIDDEN_DIM), (_HIDDEN_DIM, _WORKSPACE_DIM),
            (256, 512), (512, 256)]
    for (ns, kd, vd, *_t) in _RUNGS:
        gd = _WORKSPACE_DIM + kd + 2 * vd + 5
        plan += [(256, 512), (512, kd), (256, 512), (512, vd)]
        plan += [(gd, _GATE_HID)] * 3
        plan += [(vd, _MEMORY_TOKEN_DIM), (vd, _MEMORY_TOKEN_DIM),
                 (vd, 512), (512, _MEMORY_TOKEN_DIM)]
    return plan


_PLAN = _big_plan()
_N_BIG = len(_PLAN)


def _gelu(x):
    return jax.nn.gelu(x)


def _ln(x, g, b):
    m = x.mean(-1, keepdims=True)
    v = ((x - m) ** 2).mean(-1, keepdims=True)
    return (x - m) / jnp.sqrt(v + 1e-5) * g + b


def _dot(x, w):
    return jnp.dot(x, w, preferred_element_type=jnp.float32)


def _body(*args):
    h_ref, m_ref = args[0], args[1]
    wrefs = args[2:2 + _N_BIG]
    ctx_ref, mt_ref = args[2 + _N_BIG], args[3 + _N_BIG]
    acc_ref = args[4 + _N_BIG]
    vrefs = args[5 + _N_BIG:5 + _N_BIG + _N_BIG]
    sems = args[5 + _N_BIG + _N_BIG]

    i = pl.program_id(0)

    def copy(c):
        return pltpu.make_async_copy(wrefs[c], vrefs[c], sems.at[c])

    @pl.when(i == 0)
    def _init():
        acc_ref[...] = jnp.zeros_like(acc_ref)
        for c in range(_N_BIG):
            copy(c).start()
        copy(1).wait()

    hb = h_ref[...]
    mb = m_ref[:, pl.ds(i * _CHUNK, _CHUNK)].astype(jnp.float32)
    cs = jnp.sum(hb * mb[:, :, None], axis=1)
    acc_ref[...] += _dot(cs, vrefs[1][:_HIDDEN_DIM])

    @pl.when(i == _NSTEP - 1)
    def _epilogue():
        for c in range(_N_BIG):
            if c != 1:
                copy(c).wait()

        pk = vrefs[0]

        def pr(name):
            off, w = _OFFS[name]
            return pk[:, off:off + w]

        it = iter(vrefs[1:])

        def nxt():
            return next(it)[...]

        denom = jnp.maximum(
            jnp.sum(m_ref[...].astype(jnp.float32), axis=1, keepdims=True), 1.0)
        last = hb[:, -1, :]

        ev_w1, ev_w2 = nxt(), nxt()
        h1 = _gelu(acc_ref[...] / denom +
                   _dot(last, ev_w1[_HIDDEN_DIM:]) + pr("ev_b1"))
        ctx = _dot(h1, ev_w2) + pr("ev_b2")

        lv_w1, lv_w2 = nxt(), nxt()
        lv = _dot(_gelu(_dot(ctx, lv_w1) + pr("lv_b1")), lv_w2) + pr("lv_b2")

        def col_lin(wname, bname):
            w = pr(wname)
            z = (jnp.sum(ctx * w[:, :_WORKSPACE_DIM], axis=-1, keepdims=True) +
                 jnp.sum(lv * w[:, _WORKSPACE_DIM:], axis=-1, keepdims=True))
            return jax.nn.sigmoid(z + pr(bname))

        wp = col_lin("lw", "lw_b")
        cp_ = col_lin("lc", "lc_b")

        ctx_ref[...] = ctx
        mt_ref[...] = jnp.zeros_like(mt_ref)

        base = 0
        for r, (ns, kd, vd, rt, st, pt) in enumerate(_RUNGS):
            k_w1, k_w2, v_w1, v_w2 = nxt(), nxt(), nxt(), nxt()
            ck = _dot(_gelu(_dot(ctx, k_w1) + pr(f"r{r}_k_b1")), k_w2) \
                + pr(f"r{r}_k_b2")
            ck = ck / jnp.maximum(
                jnp.sqrt(jnp.sum(ck * ck, axis=-1, keepdims=True)), 1e-6)
            cv = _dot(_gelu(_dot(ctx, v_w1) + pr(f"r{r}_v_b1")), v_w2) \
                + pr(f"r{r}_v_b2")

            o_ck = _WORKSPACE_DIM
            o_cv = o_ck + kd
            o_mv = o_cv + vd
            o_sc = o_mv + vd
            probs = []
            for g in range(3):
                gw = nxt()
                gh = (_dot(ctx, gw[:o_ck]) +
                      _dot(ck, gw[o_ck:o_cv]) +
                      _dot(cv, gw[o_cv:o_mv]) +
                      _CAD_PRIOR * gw[o_sc + 1:o_sc + 2] +
                      gw[o_sc + 2:o_sc + 3] +
                      wp * gw[o_sc + 3:o_sc + 4] +
                      cp_ * gw[o_sc + 4:o_sc + 5] +
                      pr(f"r{r}g{g}_b1"))
                z = jnp.sum(_gelu(gh) * pr(f"r{r}g{g}_w2"),
                            axis=-1, keepdims=True)
                probs.append(jax.nn.sigmoid(z + pr(f"r{r}g{g}_b2")))
            rm = jax.nn.sigmoid((probs[0] - rt) / _TEMPERATURE)
            sm = jax.nn.sigmoid((probs[1] - st) / _TEMPERATURE)
            pm = jax.nn.sigmoid((probs[2] - pt) / _TEMPERATURE)

            summary = cv * (rm + sm - rm * sm)
            sp_w, st_w, ro_w1, ro_w2 = nxt(), nxt(), nxt(), nxt()
            promoted = pm * _ln(_dot(summary, sp_w) + pr(f"r{r}_sp_b"),
                                pr(f"r{r}_sp_g"), pr(f"r{r}_sp_bb"))
            tok0 = _ln(_dot(summary, st_w) + pr(f"r{r}_st_b"),
                       pr(f"r{r}_st_g"), pr(f"r{r}_st_bb")) \
                * jnp.maximum(sm, rm)
            read = _dot(_gelu(_dot(summary, ro_w1) + pr(f"r{r}_ro_b1")),
                        ro_w2) + pr(f"r{r}_ro_b2")

            mt_ref[:, base, :] = tok0
            mt_ref[:, base + ns, :] = read
            mt_ref[:, base + ns + 1, :] = promoted
            base += ns + 2


def _pack_and_list(params):
    rungs = params["rungs"]
    src = {
        "ev_b1": params["evidence"]["l1"]["b"],
        "ev_b2": params["evidence"]["l2"]["b"],
        "lv_b1": params["ledger_value"]["l1"]["b"],
        "lv_b2": params["ledger_value"]["l2"]["b"],
        "lw": params["ledger_write"]["w"].reshape(512),
        "lw_b": params["ledger_write"]["b"],
        "lc": params["ledger_contra"]["w"].reshape(512),
        "lc_b": params["ledger_contra"]["b"],
    }
    for r, rp in enumerate(rungs):
        src[f"r{r}_sp_b"] = rp["summary_proj"]["lin"]["b"]
        src[f"r{r}_sp_g"] = rp["summary_proj"]["ln"]["g"]
        src[f"r{r}_sp_bb"] = rp["summary_proj"]["ln"]["b"]
        src[f"r{r}_st_b"] = rp["slot_token_proj"]["lin"]["b"]
        src[f"r{r}_st_g"] = rp["slot_token_proj"]["ln"]["g"]
        src[f"r{r}_st_bb"] = rp["slot_token_proj"]["ln"]["b"]
        src[f"r{r}_ro_b2"] = rp["readout"]["l2"]["b"]
        src[f"r{r}_k_b1"] = rp["key"]["l1"]["b"]
        src[f"r{r}_k_b2"] = rp["key"]["l2"]["b"]
        src[f"r{r}_v_b1"] = rp["value"]["l1"]["b"]
        src[f"r{r}_v_b2"] = rp["value"]["l2"]["b"]
        src[f"r{r}_ro_b1"] = rp["readout"]["l1"]["b"]
        for g, gname in enumerate(("refresh", "spawn", "promote")):
            src[f"r{r}g{g}_b1"] = rp[gname]["l1"]["b"]
            src[f"r{r}g{g}_w2"] = rp[gname]["l2"]["w"].reshape(_GATE_HID)
            src[f"r{r}g{g}_b2"] = rp[gname]["l2"]["b"]

    flat = jnp.concatenate([src[n] for (n, _w) in _pack_order()])

    bigs = [flat.reshape(1, _PACK_LEN),
            params["evidence"]["l1"]["w"], params["evidence"]["l2"]["w"],
            params["ledger_value"]["l1"]["w"], params["ledger_value"]["l2"]["w"]]
    for rp in rungs:
        bigs += [rp["key"]["l1"]["w"], rp["key"]["l2"]["w"],
                 rp["value"]["l1"]["w"], rp["value"]["l2"]["w"],
                 rp["refresh"]["l1"]["w"], rp["spawn"]["l1"]["w"],
                 rp["promote"]["l1"]["w"],
                 rp["summary_proj"]["lin"]["w"], rp["slot_token_proj"]["lin"]["w"],
                 rp["readout"]["l1"]["w"], rp["readout"]["l2"]["w"]]
    return bigs


def kernel(hidden, attention_mask, params):
    B, S, D = hidden.shape
    bigs = _pack_and_list(params)

    n_tokens = sum(ns + 2 for (ns, *_rest) in _RUNGS)

    in_specs = [
        pl.BlockSpec((B, _CHUNK, D), lambda i: (0, i, 0)),
        pl.BlockSpec((B, S), lambda i: (0, 0)),
    ]
    in_specs += [pl.BlockSpec(memory_space=pltpu.MemorySpace.HBM)
                 for _ in bigs]

    scratch = [pltpu.VMEM((B, D), jnp.float32)]
    scratch += [pltpu.VMEM(shp, jnp.float32) for shp in _PLAN]
    scratch += [pltpu.SemaphoreType.DMA((_N_BIG,))]

    ctx, mt = pl.pallas_call(
        _body,
        grid=(S // _CHUNK,),
        in_specs=in_specs,
        out_specs=[
            pl.BlockSpec((B, _WORKSPACE_DIM), lambda i: (0, 0)),
            pl.BlockSpec((B, n_tokens, _MEMORY_TOKEN_DIM), lambda i: (0, 0, 0)),
        ],
        out_shape=[
            jax.ShapeDtypeStruct((B, _WORKSPACE_DIM), jnp.float32),
            jax.ShapeDtypeStruct((B, n_tokens, _MEMORY_TOKEN_DIM), jnp.float32),
        ],
        scratch_shapes=scratch,
    )(hidden, attention_mask, *bigs)
    return ctx, mt


# speedup vs baseline: 1.0227x; 1.0227x over previous
import math

import jax
import jax.numpy as jnp
from jax.experimental import pallas as pl
from jax.experimental.pallas import tpu as pltpu

_HIDDEN_DIM = 1024
_WORKSPACE_DIM = 256
_MEMORY_TOKEN_DIM = 1024
_TEMPERATURE = 0.25
_RUNGS = [
    (8, 96, 192, 0.55, 0.6, 0.5),
    (6, 128, 256, 0.55, 0.6, 0.5),
    (4, 160, 320, 0.55, 0.6, 0.5),
]
_CAD_PRIOR = 1.0 / (1.0 + math.exp(1.0))

_CHUNK = 512
_NSTEP = 4096 // _CHUNK
_GATE_HID = 384


def _pack_order():
    items = [("ev_b1", 1024)]
    for r in range(len(_RUNGS)):
        for n in ("sp_b", "sp_g", "sp_bb", "st_b", "st_g", "st_bb", "ro_b2"):
            items.append((f"r{r}_{n}", 1024))
    items += [("lv_b1", 512), ("lw", 512), ("lc", 512)]
    for r in range(len(_RUNGS)):
        for n in ("k_b1", "v_b1", "ro_b1"):
            items.append((f"r{r}_{n}", 512))
    for r in range(len(_RUNGS)):
        for g in range(3):
            items += [(f"r{r}g{g}_b1", _GATE_HID), (f"r{r}g{g}_w2", _GATE_HID)]
    items += [("ev_b2", 256), ("lv_b2", 256), ("lw_b", 1), ("lc_b", 1)]
    for r in range(len(_RUNGS)):
        for g in range(3):
            items.append((f"r{r}g{g}_b2", 1))
    for r, (_ns, kd, vd, *_t) in enumerate(_RUNGS):
        items += [(f"r{r}_k_b2", kd), (f"r{r}_v_b2", vd)]
    return items


_OFFS = {}
_o = 0
for _n, _w in _pack_order():
    _OFFS[_n] = (_o, _w)
    _o += _w
_PACK_LEN = _o


def _big_plan():
    plan = [(1, _PACK_LEN),
            (2 * _HIDDEN_DIM, _HIDDEN_DIM), (_HIDDEN_DIM, _WORKSPACE_DIM),
            (256, 512), (512, 256)]
    for (ns, kd, vd, *_t) in _RUNGS:
        gd = _WORKSPACE_DIM + kd + 2 * vd + 5
        plan += [(256, 512), (512, kd), (256, 512), (512, vd)]
        plan += [(gd, _GATE_HID)] * 3
        plan += [(vd, _MEMORY_TOKEN_DIM), (vd, _MEMORY_TOKEN_DIM),
                 (vd, 512), (512, _MEMORY_TOKEN_DIM)]
    return plan


_PLAN = _big_plan()
_N_BIG = len(_PLAN)


def _gelu(x):
    return jax.nn.gelu(x)


def _ln(x, g, b):
    m = x.mean(-1, keepdims=True)
    v = ((x - m) ** 2).mean(-1, keepdims=True)
    return (x - m) / jnp.sqrt(v + 1e-5) * g + b


def _dot(x, w):
    return jnp.dot(x, w, preferred_element_type=jnp.float32)


def _body(*args):
    h_ref, m_ref = args[0], args[1]
    wrefs = args[2:2 + _N_BIG]
    ctx_ref, mt_ref = args[2 + _N_BIG], args[3 + _N_BIG]
    acc_ref = args[4 + _N_BIG]
    vrefs = args[5 + _N_BIG:5 + _N_BIG + _N_BIG]
    sems = args[5 + _N_BIG + _N_BIG]

    i = pl.program_id(0)

    def copy(c):
        return pltpu.make_async_copy(wrefs[c], vrefs[c], sems.at[c])

    @pl.when(i == 0)
    def _init():
        acc_ref[...] = jnp.zeros_like(acc_ref)
        for c in range(_N_BIG):
            copy(c).start()

    hb = h_ref[...]
    mb = m_ref[:, pl.ds(i * _CHUNK, _CHUNK)].astype(jnp.float32)
    acc_ref[...] += jnp.sum(hb * mb[:, :, None], axis=1)

    @pl.when(i == _NSTEP - 1)
    def _epilogue():
        for c in range(_N_BIG):
            copy(c).wait()

        pk = vrefs[0]

        def pr(name):
            off, w = _OFFS[name]
            return pk[:, off:off + w]

        it = iter(vrefs[1:])

        def nxt():
            return next(it)[...]

        denom = jnp.maximum(
            jnp.sum(m_ref[...].astype(jnp.float32), axis=1, keepdims=True), 1.0)
        pooled = acc_ref[...] / denom
        last = hb[:, -1, :]

        ev_w1, ev_w2 = nxt(), nxt()
        h1 = _gelu(_dot(pooled, ev_w1[:_HIDDEN_DIM]) +
                   _dot(last, ev_w1[_HIDDEN_DIM:]) + pr("ev_b1"))
        ctx = _dot(h1, ev_w2) + pr("ev_b2")

        lv_w1, lv_w2 = nxt(), nxt()
        lv = _dot(_gelu(_dot(ctx, lv_w1) + pr("lv_b1")), lv_w2) + pr("lv_b2")

        def col_lin(wname, bname):
            w = pr(wname)
            z = (jnp.sum(ctx * w[:, :_WORKSPACE_DIM], axis=-1, keepdims=True) +
                 jnp.sum(lv * w[:, _WORKSPACE_DIM:], axis=-1, keepdims=True))
            return jax.nn.sigmoid(z + pr(bname))

        wp = col_lin("lw", "lw_b")
        cp_ = col_lin("lc", "lc_b")

        ctx_ref[...] = ctx
        mt_ref[...] = jnp.zeros_like(mt_ref)

        base = 0
        for r, (ns, kd, vd, rt, st, pt) in enumerate(_RUNGS):
            k_w1, k_w2, v_w1, v_w2 = nxt(), nxt(), nxt(), nxt()
            ck = _dot(_gelu(_dot(ctx, k_w1) + pr(f"r{r}_k_b1")), k_w2) \
                + pr(f"r{r}_k_b2")
            ck = ck / jnp.maximum(
                jnp.sqrt(jnp.sum(ck * ck, axis=-1, keepdims=True)), 1e-6)
            cv = _dot(_gelu(_dot(ctx, v_w1) + pr(f"r{r}_v_b1")), v_w2) \
                + pr(f"r{r}_v_b2")

            o_ck = _WORKSPACE_DIM
            o_cv = o_ck + kd
            o_mv = o_cv + vd
            o_sc = o_mv + vd
            probs = []
            for g in range(3):
                gw = nxt()
                gh = (_dot(ctx, gw[:o_ck]) +
                      _dot(ck, gw[o_ck:o_cv]) +
                      _dot(cv, gw[o_cv:o_mv]) +
                      _CAD_PRIOR * gw[o_sc + 1:o_sc + 2] +
                      gw[o_sc + 2:o_sc + 3] +
                      wp * gw[o_sc + 3:o_sc + 4] +
                      cp_ * gw[o_sc + 4:o_sc + 5] +
                      pr(f"r{r}g{g}_b1"))
                z = jnp.sum(_gelu(gh) * pr(f"r{r}g{g}_w2"),
                            axis=-1, keepdims=True)
                probs.append(jax.nn.sigmoid(z + pr(f"r{r}g{g}_b2")))
            rm = jax.nn.sigmoid((probs[0] - rt) / _TEMPERATURE)
            sm = jax.nn.sigmoid((probs[1] - st) / _TEMPERATURE)
            pm = jax.nn.sigmoid((probs[2] - pt) / _TEMPERATURE)

            summary = cv * (rm + sm - rm * sm)
            sp_w, st_w, ro_w1, ro_w2 = nxt(), nxt(), nxt(), nxt()
            promoted = pm * _ln(_dot(summary, sp_w) + pr(f"r{r}_sp_b"),
                                pr(f"r{r}_sp_g"), pr(f"r{r}_sp_bb"))
            tok0 = _ln(_dot(summary, st_w) + pr(f"r{r}_st_b"),
                       pr(f"r{r}_st_g"), pr(f"r{r}_st_bb")) \
                * jnp.maximum(sm, rm)
            read = _dot(_gelu(_dot(summary, ro_w1) + pr(f"r{r}_ro_b1")),
                        ro_w2) + pr(f"r{r}_ro_b2")

            mt_ref[:, base, :] = tok0
            mt_ref[:, base + ns, :] = read
            mt_ref[:, base + ns + 1, :] = promoted
            base += ns + 2


def _pack_and_list(params):
    rungs = params["rungs"]
    src = {
        "ev_b1": params["evidence"]["l1"]["b"],
        "ev_b2": params["evidence"]["l2"]["b"],
        "lv_b1": params["ledger_value"]["l1"]["b"],
        "lv_b2": params["ledger_value"]["l2"]["b"],
        "lw": params["ledger_write"]["w"].reshape(512),
        "lw_b": params["ledger_write"]["b"],
        "lc": params["ledger_contra"]["w"].reshape(512),
        "lc_b": params["ledger_contra"]["b"],
    }
    for r, rp in enumerate(rungs):
        src[f"r{r}_sp_b"] = rp["summary_proj"]["lin"]["b"]
        src[f"r{r}_sp_g"] = rp["summary_proj"]["ln"]["g"]
        src[f"r{r}_sp_bb"] = rp["summary_proj"]["ln"]["b"]
        src[f"r{r}_st_b"] = rp["slot_token_proj"]["lin"]["b"]
        src[f"r{r}_st_g"] = rp["slot_token_proj"]["ln"]["g"]
        src[f"r{r}_st_bb"] = rp["slot_token_proj"]["ln"]["b"]
        src[f"r{r}_ro_b2"] = rp["readout"]["l2"]["b"]
        src[f"r{r}_k_b1"] = rp["key"]["l1"]["b"]
        src[f"r{r}_k_b2"] = rp["key"]["l2"]["b"]
        src[f"r{r}_v_b1"] = rp["value"]["l1"]["b"]
        src[f"r{r}_v_b2"] = rp["value"]["l2"]["b"]
        src[f"r{r}_ro_b1"] = rp["readout"]["l1"]["b"]
        for g, gname in enumerate(("refresh", "spawn", "promote")):
            src[f"r{r}g{g}_b1"] = rp[gname]["l1"]["b"]
            src[f"r{r}g{g}_w2"] = rp[gname]["l2"]["w"].reshape(_GATE_HID)
            src[f"r{r}g{g}_b2"] = rp[gname]["l2"]["b"]

    flat = jnp.concatenate([src[n] for (n, _w) in _pack_order()])

    bigs = [flat.reshape(1, _PACK_LEN),
            params["evidence"]["l1"]["w"], params["evidence"]["l2"]["w"],
            params["ledger_value"]["l1"]["w"], params["ledger_value"]["l2"]["w"]]
    for rp in rungs:
        bigs += [rp["key"]["l1"]["w"], rp["key"]["l2"]["w"],
                 rp["value"]["l1"]["w"], rp["value"]["l2"]["w"],
                 rp["refresh"]["l1"]["w"], rp["spawn"]["l1"]["w"],
                 rp["promote"]["l1"]["w"],
                 rp["summary_proj"]["lin"]["w"], rp["slot_token_proj"]["lin"]["w"],
                 rp["readout"]["l1"]["w"], rp["readout"]["l2"]["w"]]
    return bigs


def kernel(hidden, attention_mask, params):
    B, S, D = hidden.shape
    bigs = _pack_and_list(params)

    n_tokens = sum(ns + 2 for (ns, *_rest) in _RUNGS)

    in_specs = [
        pl.BlockSpec((B, _CHUNK, D), lambda i: (0, i, 0)),
        pl.BlockSpec((B, S), lambda i: (0, 0)),
    ]
    in_specs += [pl.BlockSpec(memory_space=pltpu.MemorySpace.HBM)
                 for _ in bigs]

    scratch = [pltpu.VMEM((B, D), jnp.float32)]
    scratch += [pltpu.VMEM(shp, jnp.float32) for shp in _PLAN]
    scratch += [pltpu.SemaphoreType.DMA((_N_BIG,))]

    ctx, mt = pl.pallas_call(
        _body,
        grid=(S // _CHUNK,),
        in_specs=in_specs,
        out_specs=[
            pl.BlockSpec((B, _WORKSPACE_DIM), lambda i: (0, 0)),
            pl.BlockSpec((B, n_tokens, _MEMORY_TOKEN_DIM), lambda i: (0, 0, 0)),
        ],
        out_shape=[
            jax.ShapeDtypeStruct((B, _WORKSPACE_DIM), jnp.float32),
            jax.ShapeDtypeStruct((B, n_tokens, _MEMORY_TOKEN_DIM), jnp.float32),
        ],
        scratch_shapes=scratch,
    )(hidden, attention_mask, *bigs)
    return ctx, mt
